# baseline (speedup 1.0000x reference)
import functools
import math

import jax
import jax.numpy as jnp
from jax import lax
from jax.experimental import pallas as pl
from jax.experimental.pallas import tpu as pltpu
from jax.experimental.pallas import tpu_sc as plsc

D_MODEL = 768
SCALE = math.sqrt(float(D_MODEL))
B = 4 * 8192

_INFO = plsc.get_sparse_core_info()
NC = _INFO.num_cores
NS = _INFO.num_subcores
L = _INFO.num_lanes
NW = NC * NS
BPW = B // NW
CH = 64
NCHUNK = BPW // CH
COLS = D_MODEL // L


def _emb_body(x_hbm, tab_hbm, out_hbm, idx_v, buf0, buf1, sg0, sg1, ss0, ss1):
    wid = lax.axis_index("s") * NC + lax.axis_index("c")
    base = wid * BPW

    pltpu.sync_copy(x_hbm.at[wid], idx_v)

    bufs = (buf0, buf1)
    gsems = (sg0, sg1)
    ssems = (ss0, ss1)
    gather = [None, None]
    store = [None, None]

    gather[0] = pltpu.async_copy(tab_hbm.at[idx_v.at[0]], buf0, sg0)

    for j in range(NCHUNK):
        p = j & 1
        o = p ^ 1
        if j + 1 < NCHUNK:
            if store[o] is not None:
                store[o].wait()
                store[o] = None
            gather[o] = pltpu.async_copy(
                tab_hbm.at[idx_v.at[j + 1]], bufs[o], gsems[o]
            )
        gather[p].wait()
        buf = bufs[p]

        def row_body(r, carry):
            def col_body(c, carry2):
                sl = pl.ds(c * L, L)
                buf[r, sl] = buf[r, sl] * SCALE
                return carry2

            return lax.fori_loop(0, COLS, col_body, carry)

        lax.fori_loop(0, CH, row_body, 0)

        store[p] = pltpu.async_copy(
            buf, out_hbm.at[pl.ds(base + j * CH, CH)], ssems[p]
        )

    for h in store:
        if h is not None:
            h.wait()


def kernel(x, emb_weight):
    xf = x.reshape(NW, NCHUNK, CH).astype(jnp.int32)
    mesh = plsc.VectorSubcoreMesh(core_axis_name="c", subcore_axis_name="s")
    out = pl.kernel(
        _emb_body,
        out_type=jax.ShapeDtypeStruct((B, D_MODEL), jnp.float32),
        mesh=mesh,
        scratch_types=[
            pltpu.VMEM((NCHUNK, CH), jnp.int32),
            pltpu.VMEM((CH, D_MODEL), jnp.float32),
            pltpu.VMEM((CH, D_MODEL), jnp.float32),
            pltpu.SemaphoreType.DMA,
            pltpu.SemaphoreType.DMA,
            pltpu.SemaphoreType.DMA,
            pltpu.SemaphoreType.DMA,
        ],
    )(xf, emb_weight)
    return out.reshape(x.shape[0], x.shape[1], D_MODEL)


# speedup vs baseline: 3.0214x; 3.0214x over previous
import functools
import math

import jax
import jax.numpy as jnp
from jax import lax
from jax.experimental import pallas as pl
from jax.experimental.pallas import tpu as pltpu
from jax.experimental.pallas import tpu_sc as plsc

D_MODEL = 768
SCALE = math.sqrt(float(D_MODEL))
B = 4 * 8192

_INFO = plsc.get_sparse_core_info()
NC = _INFO.num_cores
NS = _INFO.num_subcores
L = _INFO.num_lanes
NW = NC * NS
BPW = B // NW
CH = 64
NCHUNK = BPW // CH
COLS = D_MODEL // L


def _emb_body(x_hbm, tab_hbm, out_hbm, idx_v, buf0, buf1, sg0, sg1, ss0, ss1):
    wid = lax.axis_index("s") * NC + lax.axis_index("c")
    base = wid * BPW

    pltpu.sync_copy(x_hbm.at[wid], idx_v)

    bufs = (buf0, buf1)
    gsems = (sg0, sg1)
    ssems = (ss0, ss1)
    gather = [None, None]
    store = [None, None]

    gather[0] = pltpu.async_copy(tab_hbm.at[idx_v.at[0]], buf0, sg0)

    for j in range(NCHUNK):
        p = j & 1
        o = p ^ 1
        if j + 1 < NCHUNK:
            if store[o] is not None:
                store[o].wait()
                store[o] = None
            gather[o] = pltpu.async_copy(
                tab_hbm.at[idx_v.at[j + 1]], bufs[o], gsems[o]
            )
        gather[p].wait()
        buf = bufs[p]

        @plsc.parallel_loop(0, CH, step=1, unroll=2)
        def _scale_row(r):
            for c in range(COLS):
                sl = pl.ds(c * L, L)
                buf[r, sl] = buf[r, sl] * SCALE

        store[p] = pltpu.async_copy(
            buf, out_hbm.at[pl.ds(base + j * CH, CH)], ssems[p]
        )

    for h in store:
        if h is not None:
            h.wait()


def kernel(x, emb_weight):
    xf = x.reshape(NW, NCHUNK, CH).astype(jnp.int32)
    mesh = plsc.VectorSubcoreMesh(core_axis_name="c", subcore_axis_name="s")
    out = pl.kernel(
        _emb_body,
        out_type=jax.ShapeDtypeStruct((B, D_MODEL), jnp.float32),
        mesh=mesh,
        scratch_types=[
            pltpu.VMEM((NCHUNK, CH), jnp.int32),
            pltpu.VMEM((CH, D_MODEL), jnp.float32),
            pltpu.VMEM((CH, D_MODEL), jnp.float32),
            pltpu.SemaphoreType.DMA,
            pltpu.SemaphoreType.DMA,
            pltpu.SemaphoreType.DMA,
            pltpu.SemaphoreType.DMA,
        ],
    )(xf, emb_weight)
    return out.reshape(x.shape[0], x.shape[1], D_MODEL)


# speedup vs baseline: 3.0461x; 1.0082x over previous
import functools
import math

import jax
import jax.numpy as jnp
from jax import lax
from jax.experimental import pallas as pl
from jax.experimental.pallas import tpu as pltpu
from jax.experimental.pallas import tpu_sc as plsc

D_MODEL = 768
SCALE = math.sqrt(float(D_MODEL))
B = 4 * 8192

_INFO = plsc.get_sparse_core_info()
NC = _INFO.num_cores
NS = _INFO.num_subcores
L = _INFO.num_lanes
NW = NC * NS
BPW = B // NW
CH = 32
NCHUNK = BPW // CH
COLS = D_MODEL // L


NB = 4


def _emb_body(x_hbm, tab_hbm, out_hbm, idx_v,
              buf0, buf1, buf2, buf3,
              sg0, sg1, sg2, sg3, ss0, ss1, ss2, ss3):
    wid = lax.axis_index("s") * NC + lax.axis_index("c")
    base = wid * BPW

    pltpu.sync_copy(x_hbm.at[wid], idx_v)

    bufs = (buf0, buf1, buf2, buf3)
    gsems = (sg0, sg1, sg2, sg3)
    ssems = (ss0, ss1, ss2, ss3)
    gather = [None] * NB
    store = [None] * NB

    gather[0] = pltpu.async_copy(tab_hbm.at[idx_v.at[0]], bufs[0], gsems[0])

    for j in range(NCHUNK):
        p = j % NB
        if j + 1 < NCHUNK:
            o = (j + 1) % NB
            if store[o] is not None:
                store[o].wait()
                store[o] = None
            gather[o] = pltpu.async_copy(
                tab_hbm.at[idx_v.at[j + 1]], bufs[o], gsems[o]
            )
        gather[p].wait()
        buf = bufs[p]

        @plsc.parallel_loop(0, CH, step=1, unroll=1)
        def _scale_row(r):
            for c in range(COLS):
                sl = pl.ds(c * L, L)
                buf[r, sl] = buf[r, sl] * SCALE

        store[p] = pltpu.async_copy(
            buf, out_hbm.at[pl.ds(base + j * CH, CH)], ssems[p]
        )

    for h in store:
        if h is not None:
            h.wait()


def kernel(x, emb_weight):
    xf = x.reshape(NW, NCHUNK, CH).astype(jnp.int32)
    mesh = plsc.VectorSubcoreMesh(core_axis_name="c", subcore_axis_name="s")
    out = pl.kernel(
        _emb_body,
        out_type=jax.ShapeDtypeStruct((B, D_MODEL), jnp.float32),
        mesh=mesh,
        scratch_types=(
            [pltpu.VMEM((NCHUNK, CH), jnp.int32)]
            + [pltpu.VMEM((CH, D_MODEL), jnp.float32)] * NB
            + [pltpu.SemaphoreType.DMA] * (2 * NB)
        ),
    )(xf, emb_weight)
    return out.reshape(x.shape[0], x.shape[1], D_MODEL)


# speedup vs baseline: 3.0736x; 1.0090x over previous
import functools
import math

import jax
import jax.numpy as jnp
from jax import lax
from jax.experimental import pallas as pl
from jax.experimental.pallas import tpu as pltpu
from jax.experimental.pallas import tpu_sc as plsc

D_MODEL = 768
SCALE = math.sqrt(float(D_MODEL))
B = 4 * 8192

_INFO = plsc.get_sparse_core_info()
NC = _INFO.num_cores
NS = _INFO.num_subcores
L = _INFO.num_lanes
NW = NC * NS
BPW = B // NW
CH = 32
NCHUNK = BPW // CH
COLS = D_MODEL // L


NB = 4


def _emb_body(x_hbm, tab_hbm, out_hbm, idx_v,
              buf0, buf1, buf2, buf3,
              sg0, sg1, sg2, sg3, ss0, ss1, ss2, ss3):
    wid = lax.axis_index("s") * NC + lax.axis_index("c")
    base = wid * BPW

    pltpu.sync_copy(x_hbm.at[wid], idx_v)

    bufs = (buf0, buf1, buf2, buf3)
    gsems = (sg0, sg1, sg2, sg3)
    ssems = (ss0, ss1, ss2, ss3)
    gather = [None] * NB
    store = [None] * NB

    AHEAD = 2
    for j in range(AHEAD):
        gather[j] = pltpu.async_copy(tab_hbm.at[idx_v.at[j]], bufs[j], gsems[j])

    for j in range(NCHUNK):
        p = j % NB
        if j + AHEAD < NCHUNK:
            o = (j + AHEAD) % NB
            if store[o] is not None:
                store[o].wait()
                store[o] = None
            gather[o] = pltpu.async_copy(
                tab_hbm.at[idx_v.at[j + AHEAD]], bufs[o], gsems[o]
            )
        gather[p].wait()
        buf = bufs[p]

        @plsc.parallel_loop(0, CH, step=1, unroll=1)
        def _scale_row(r):
            for c in range(COLS):
                sl = pl.ds(c * L, L)
                buf[r, sl] = buf[r, sl] * SCALE

        store[p] = pltpu.async_copy(
            buf, out_hbm.at[pl.ds(base + j * CH, CH)], ssems[p]
        )

    for h in store:
        if h is not None:
            h.wait()


def kernel(x, emb_weight):
    xf = x.reshape(NW, NCHUNK, CH).astype(jnp.int32)
    mesh = plsc.VectorSubcoreMesh(core_axis_name="c", subcore_axis_name="s")
    out = pl.kernel(
        _emb_body,
        out_type=jax.ShapeDtypeStruct((B, D_MODEL), jnp.float32),
        mesh=mesh,
        scratch_types=(
            [pltpu.VMEM((NCHUNK, CH), jnp.int32)]
            + [pltpu.VMEM((CH, D_MODEL), jnp.float32)] * NB
            + [pltpu.SemaphoreType.DMA] * (2 * NB)
        ),
    )(xf, emb_weight)
    return out.reshape(x.shape[0], x.shape[1], D_MODEL)
